# baseline (speedup 1.0000x reference)
import functools

import jax
import jax.numpy as jnp
from jax import lax
from jax.experimental import pallas as pl
from jax.experimental.pallas import tpu as pltpu
from jax.experimental.pallas import tpu_sc as plsc

_NUM_CORES = 2
_NUM_SUBCORES = 16
_NW = _NUM_CORES * _NUM_SUBCORES
_L = 16
_LANES = 128
_SUB = 8


@functools.cache
def _build_gather_t(S, J, D):
    assert S % (_NW * _LANES) == 0 and D % _SUB == 0
    assert J % 2 == 0 and J >= 4
    TD = D // _SUB
    TS = S // _LANES
    nblk = TS // _NW
    ngrp = _LANES // _L
    mesh = plsc.VectorSubcoreMesh(core_axis_name="c", subcore_axis_name="s")

    def body(table_hbm, idx_hbm, p_hbm, idx2_v, idxT_v, rows_v, q_v,
             g0, g1, q0, q1):
        gsem = (g0, g1)
        qsem = (q0, q1)
        wid = lax.axis_index("s") * _NUM_CORES + lax.axis_index("c")
        iota = lax.iota(jnp.int32, _L)
        sv = [iota + _L * k for k in range(ngrp)]
        svj = [(iota + _L * k) * J for k in range(ngrp)]

        @pl.loop(0, nblk)
        def _blk(tsb):
            ts = wid * nblk + tsb
            tok0 = ts * _LANES * J

            pltpu.sync_copy(idx_hbm.at[pl.ds(tok0, _LANES * J)], idx2_v)

            @pl.loop(0, J)
            def _tr_idx(j):
                for k in range(ngrp):
                    v = plsc.load_gather(idx2_v, [svj[k] + j])
                    idxT_v[j, pl.ds(_L * k, _L)] = v

            def g_issue(j, b):
                pltpu.async_copy(
                    table_hbm.at[idxT_v.at[j]], rows_v.at[b], gsem[b])

            def g_wait(j, b):
                pltpu.make_async_copy(
                    table_hbm.at[idxT_v.at[j]], rows_v.at[b], gsem[b]).wait()

            def q_store(j, b):
                pltpu.async_copy(q_v.at[b], p_hbm.at[j, :, ts], qsem[b])

            def q_wait(j, b):
                pltpu.make_async_copy(
                    q_v.at[b], p_hbm.at[j, :, ts], qsem[b]).wait()

            def transpose(b):
                @pl.loop(0, TD)
                def _t(td):
                    for di in range(_SUB):
                        dv = jnp.full((_L,), td * _SUB + di, jnp.int32)
                        for k in range(ngrp):
                            v = plsc.load_gather(rows_v.at[b], [sv[k], dv])
                            q_v[b, td, di, pl.ds(_L * k, _L)] = v

            g_issue(0, 0)
            g_issue(1, 1)
            for j0 in range(2):
                g_wait(j0, j0)
                transpose(j0)
                g_issue(j0 + 2, j0)
                q_store(j0, j0)

            @pl.loop(2, J - 2, step=2)
            def _main(jv):
                for u in range(2):
                    j = jv + u
                    g_wait(j, u)
                    q_wait(j - 2, u)
                    transpose(u)
                    g_issue(j + 2, u)
                    q_store(j, u)

            for j0 in (J - 2, J - 1):
                b = j0 % 2
                g_wait(j0, b)
                q_wait(j0 - 2, b)
                transpose(b)
                q_store(j0, b)
            q_wait(J - 2, 0)
            q_wait(J - 1, 1)

    return pl.kernel(
        body,
        out_type=jax.ShapeDtypeStruct((J, TD, TS, _SUB, _LANES), jnp.float32),
        mesh=mesh,
        compiler_params=pltpu.CompilerParams(
            use_tc_tiling_on_sc=False, needs_layout_passes=False),
        scratch_types=[
            pltpu.VMEM((_LANES * J,), jnp.int32),
            pltpu.VMEM((J, _LANES), jnp.int32),
            pltpu.VMEM((2, _LANES, D), jnp.float32),
            pltpu.VMEM((2, TD, _SUB, _LANES), jnp.float32),
        ] + [pltpu.SemaphoreType.DMA] * 4,
    )


_CHUNK = 800
_NBUF = 2


@functools.cache
def _build_gather_flat(B, D):
    assert B % (_NW * _CHUNK * _NBUF) == 0
    b_per_w = B // _NW
    n_chunks = b_per_w // _CHUNK
    mesh = plsc.VectorSubcoreMesh(core_axis_name="c", subcore_axis_name="s")

    def body(table_hbm, idx_hbm, out_hbm, idx_v, rows_v, *sems):
        wid = lax.axis_index("s") * _NUM_CORES + lax.axis_index("c")
        base = wid * b_per_w

        def issue(g, b):
            off = base + g * _CHUNK
            pltpu.sync_copy(idx_hbm.at[pl.ds(off, _CHUNK)], idx_v.at[b])
            pltpu.async_copy(table_hbm.at[idx_v.at[b]], rows_v.at[b], sems[b])

        def drain_store(g, b):
            pltpu.make_async_copy(
                table_hbm.at[idx_v.at[b]], rows_v.at[b], sems[b]).wait()
            off = base + g * _CHUNK
            pltpu.sync_copy(rows_v.at[b], out_hbm.at[pl.ds(off, _CHUNK)])

        for b in range(_NBUF):
            issue(b, b)

        @pl.loop(0, n_chunks - _NBUF, step=_NBUF)
        def _main(go):
            for b in range(_NBUF):
                drain_store(go + b, b)
                issue(go + b + _NBUF, b)

        for b in range(_NBUF):
            drain_store(n_chunks - _NBUF + b, b)

    return pl.kernel(
        body,
        out_type=jax.ShapeDtypeStruct((B, D), jnp.float32),
        mesh=mesh,
        compiler_params=pltpu.CompilerParams(
            use_tc_tiling_on_sc=False, needs_layout_passes=False),
        scratch_types=[
            pltpu.VMEM((_NBUF, _CHUNK), jnp.int32),
            pltpu.VMEM((_NBUF, _CHUNK, D), jnp.float32),
        ] + [pltpu.SemaphoreType.DMA] * _NBUF,
    )


def kernel(token_ids, weight):
    lead = token_ids.shape
    d = weight.shape[1]
    b = 1
    for s_ in lead:
        b *= s_
    idx = token_ids.reshape((b,)).astype(jnp.int32)
    if (len(lead) == 2 and lead[0] % (_NW * _LANES) == 0 and d % _SUB == 0
            and lead[1] % 2 == 0 and lead[1] >= 4):
        s, j = lead
        p = _build_gather_t(s, j, d)(weight, idx)
        return p.transpose(2, 4, 0, 1, 3).reshape(s, j, d)
    out = _build_gather_flat(b, d)(weight, idx)
    return out.reshape(lead + (d,))


# speedup vs baseline: 1.1436x; 1.1436x over previous
import functools

import jax
import jax.numpy as jnp
from jax import lax
from jax.experimental import pallas as pl
from jax.experimental.pallas import tpu as pltpu
from jax.experimental.pallas import tpu_sc as plsc

_NUM_CORES = 2
_NUM_SUBCORES = 16
_NW = _NUM_CORES * _NUM_SUBCORES
_L = 16
_LANES = 128
_SUB = 8
_JG = 5


@functools.cache
def _build_gather_t(S, J, D):
    assert S % (_NW * _LANES) == 0 and D % _SUB == 0
    assert J % (2 * _JG) == 0
    TD = D // _SUB
    TS = S // _LANES
    nblk = TS // _NW
    ngrp = _LANES // _L
    ngl = J // _JG
    mesh = plsc.VectorSubcoreMesh(core_axis_name="c", subcore_axis_name="s")

    def body(table_hbm, idx_hbm, p_hbm, idx2_v, idxT_v, rows_v, q_v,
             g0, g1, q0, q1):
        gsem = (g0, g1)
        qsem = (q0, q1)
        wid = lax.axis_index("s") * _NUM_CORES + lax.axis_index("c")
        iota = lax.iota(jnp.int32, _L)
        sv = [iota + _L * k for k in range(ngrp)]
        svj = [(iota + _L * k) * J for k in range(ngrp)]

        @pl.loop(0, nblk)
        def _blk(tsb):
            ts = wid * nblk + tsb
            tok0 = ts * _LANES * J

            pltpu.sync_copy(idx_hbm.at[pl.ds(tok0, _LANES * J)], idx2_v)

            @pl.loop(0, J)
            def _tr_idx(j):
                for k in range(ngrp):
                    v = plsc.load_gather(idx2_v, [svj[k] + j])
                    idxT_v[pl.ds(j * _LANES + _L * k, _L)] = v

            def g_issue(grp, b):
                offs = idxT_v.at[pl.ds(grp * _JG * _LANES, _JG * _LANES)]
                pltpu.async_copy(table_hbm.at[offs], rows_v.at[b], gsem[b])

            def g_wait(grp, b):
                offs = idxT_v.at[pl.ds(grp * _JG * _LANES, _JG * _LANES)]
                pltpu.make_async_copy(
                    table_hbm.at[offs], rows_v.at[b], gsem[b]).wait()

            def q_store(j, qb):
                pltpu.async_copy(q_v.at[qb], p_hbm.at[j, :, ts], qsem[qb])

            def q_drain(j, qb):
                pltpu.make_async_copy(
                    q_v.at[qb], p_hbm.at[j, :, ts], qsem[qb]).wait()

            def transpose(b, jj, qb):
                srows = [sv[k] + jj * _LANES for k in range(ngrp)]

                @pl.loop(0, TD)
                def _t(td):
                    for di in range(_SUB):
                        dv = jnp.full((_L,), td * _SUB + di, jnp.int32)
                        vs = [
                            plsc.load_gather(rows_v.at[b], [srows[k], dv])
                            for k in range(ngrp)
                        ]
                        for k in range(ngrp):
                            q_v[qb, td, di, pl.ds(_L * k, _L)] = vs[k]

            def do_group(grp, b, first):
                g_wait(grp, b)
                for jj in range(_JG):
                    j = grp * _JG + jj
                    qb = jj % 2
                    if not (first and jj < 2):
                        q_drain(j, qb)
                    transpose(b, jj, qb)
                    q_store(j, qb)

            g_issue(0, 0)
            g_issue(1, 1)
            do_group(0, 0, True)
            g_issue(2, 0)
            do_group(1, 1, False)
            g_issue(3, 1)

            @pl.loop(2, ngl - 2, step=2)
            def _main(gv):
                for u in range(2):
                    do_group(gv + u, u, False)
                    g_issue(gv + u + 2, u)

            do_group(ngl - 2, 0, False)
            do_group(ngl - 1, 1, False)
            q_drain(0, 0)
            q_drain(1, 1)

    return pl.kernel(
        body,
        out_type=jax.ShapeDtypeStruct((J, TD, TS, _SUB, _LANES), jnp.float32),
        mesh=mesh,
        compiler_params=pltpu.CompilerParams(
            use_tc_tiling_on_sc=False, needs_layout_passes=False),
        scratch_types=[
            pltpu.VMEM((_LANES * J,), jnp.int32),
            pltpu.VMEM((_LANES * J,), jnp.int32),
            pltpu.VMEM((2, _JG * _LANES, D), jnp.float32),
            pltpu.VMEM((2, TD, _SUB, _LANES), jnp.float32),
        ] + [pltpu.SemaphoreType.DMA] * 4,
    )


_CHUNK = 800
_NBUF = 2


@functools.cache
def _build_gather_flat(B, D):
    assert B % (_NW * _CHUNK * _NBUF) == 0
    b_per_w = B // _NW
    n_chunks = b_per_w // _CHUNK
    mesh = plsc.VectorSubcoreMesh(core_axis_name="c", subcore_axis_name="s")

    def body(table_hbm, idx_hbm, out_hbm, idx_v, rows_v, *sems):
        wid = lax.axis_index("s") * _NUM_CORES + lax.axis_index("c")
        base = wid * b_per_w

        def issue(g, b):
            off = base + g * _CHUNK
            pltpu.sync_copy(idx_hbm.at[pl.ds(off, _CHUNK)], idx_v.at[b])
            pltpu.async_copy(table_hbm.at[idx_v.at[b]], rows_v.at[b], sems[b])

        def drain_store(g, b):
            pltpu.make_async_copy(
                table_hbm.at[idx_v.at[b]], rows_v.at[b], sems[b]).wait()
            off = base + g * _CHUNK
            pltpu.sync_copy(rows_v.at[b], out_hbm.at[pl.ds(off, _CHUNK)])

        for b in range(_NBUF):
            issue(b, b)

        @pl.loop(0, n_chunks - _NBUF, step=_NBUF)
        def _main(go):
            for b in range(_NBUF):
                drain_store(go + b, b)
                issue(go + b + _NBUF, b)

        for b in range(_NBUF):
            drain_store(n_chunks - _NBUF + b, b)

    return pl.kernel(
        body,
        out_type=jax.ShapeDtypeStruct((B, D), jnp.float32),
        mesh=mesh,
        compiler_params=pltpu.CompilerParams(
            use_tc_tiling_on_sc=False, needs_layout_passes=False),
        scratch_types=[
            pltpu.VMEM((_NBUF, _CHUNK), jnp.int32),
            pltpu.VMEM((_NBUF, _CHUNK, D), jnp.float32),
        ] + [pltpu.SemaphoreType.DMA] * _NBUF,
    )


def kernel(token_ids, weight):
    lead = token_ids.shape
    d = weight.shape[1]
    b = 1
    for s_ in lead:
        b *= s_
    idx = token_ids.reshape((b,)).astype(jnp.int32)
    if (len(lead) == 2 and lead[0] % (_NW * _LANES) == 0 and d % _SUB == 0
            and lead[1] % (2 * _JG) == 0):
        s, j = lead
        p = _build_gather_t(s, j, d)(weight, idx)
        return p.transpose(2, 4, 0, 1, 3).reshape(s, j, d)
    out = _build_gather_flat(b, d)(weight, idx)
    return out.reshape(lead + (d,))


# speedup vs baseline: 1.7965x; 1.5709x over previous
import functools

import jax
import jax.numpy as jnp
from jax import lax
from jax.experimental import pallas as pl
from jax.experimental.pallas import tpu as pltpu
from jax.experimental.pallas import tpu_sc as plsc

_NUM_CORES = 2
_NUM_SUBCORES = 16
_NW = _NUM_CORES * _NUM_SUBCORES
_L = 16
_LANES = 128
_SUB = 8
_JG = 5


@functools.cache
def _build_gather_t(S, J, D):
    assert S % (_NW * _LANES) == 0 and D % _SUB == 0
    assert J % (2 * _JG) == 0
    TD = D // _SUB
    TS = S // _LANES
    nblk = TS // _NW
    ngrp = _LANES // _L
    ngl = J // _JG
    mesh = plsc.VectorSubcoreMesh(core_axis_name="c", subcore_axis_name="s")

    def body(table_hbm, idx_hbm, p_hbm, idx2_v, idxT_v, rows_v, q_v,
             g0, g1, q0, q1):
        gsem = (g0, g1)
        qsem = (q0, q1)
        wid = lax.axis_index("s") * _NUM_CORES + lax.axis_index("c")
        iota = lax.iota(jnp.int32, _L)
        sv = [iota + _L * k for k in range(ngrp)]
        svj = [(iota + _L * k) * J for k in range(ngrp)]

        @pl.loop(0, nblk)
        def _blk(tsb):
            ts = wid * nblk + tsb
            tok0 = ts * _LANES * J

            pltpu.sync_copy(idx_hbm.at[pl.ds(tok0, _LANES * J)], idx2_v)

            @pl.loop(0, J)
            def _tr_idx(j):
                for k in range(ngrp):
                    v = plsc.load_gather(idx2_v, [svj[k] + j])
                    idxT_v[pl.ds(j * _LANES + _L * k, _L)] = v

            def g_issue(grp, b):
                offs = idxT_v.at[pl.ds(grp * _JG * _LANES, _JG * _LANES)]
                pltpu.async_copy(table_hbm.at[offs], rows_v.at[b], gsem[b])

            def g_wait(grp, b):
                offs = idxT_v.at[pl.ds(grp * _JG * _LANES, _JG * _LANES)]
                pltpu.make_async_copy(
                    table_hbm.at[offs], rows_v.at[b], gsem[b]).wait()

            def q_store(j, qb):
                pltpu.async_copy(
                    q_v.at[qb, :, :, pl.ds(0, _LANES)],
                    p_hbm.at[j, :, ts], qsem[qb])

            def q_drain(j, qb):
                pltpu.make_async_copy(
                    q_v.at[qb, :, :, pl.ds(0, _LANES)],
                    p_hbm.at[j, :, ts], qsem[qb]).wait()

            ndg = D // _L
            tdv = [(jnp.full((_L,), dg * _L, jnp.int32) + iota) // _SUB
                   for dg in range(ndg)]
            div = [(jnp.full((_L,), dg * _L, jnp.int32) + iota) % _SUB
                   for dg in range(ndg)]

            def transpose(b, jj, qb):
                @pl.loop(0, _LANES)
                def _t(s_in):
                    row = jj * _LANES + s_in
                    lane = jnp.full((_L,), s_in, jnp.int32)
                    for dg in range(ndg):
                        v = rows_v[b, row, pl.ds(dg * _L, _L)]
                        plsc.store_scatter(
                            q_v.at[qb], [tdv[dg], div[dg], lane], v)

            def do_group(grp, b, first):
                g_wait(grp, b)
                for jj in range(_JG):
                    j = grp * _JG + jj
                    qb = jj % 2
                    if not (first and jj < 2):
                        q_drain(j, qb)
                    transpose(b, jj, qb)
                    q_store(j, qb)

            g_issue(0, 0)
            g_issue(1, 1)
            do_group(0, 0, True)
            g_issue(2, 0)
            do_group(1, 1, False)
            g_issue(3, 1)

            @pl.loop(2, ngl - 2, step=2)
            def _main(gv):
                for u in range(2):
                    do_group(gv + u, u, False)
                    g_issue(gv + u + 2, u)

            do_group(ngl - 2, 0, False)
            do_group(ngl - 1, 1, False)
            q_drain(0, 0)
            q_drain(1, 1)

    return pl.kernel(
        body,
        out_type=jax.ShapeDtypeStruct((J, TD, TS, _SUB, _LANES), jnp.float32),
        mesh=mesh,
        compiler_params=pltpu.CompilerParams(
            use_tc_tiling_on_sc=False, needs_layout_passes=False),
        scratch_types=[
            pltpu.VMEM((_LANES * J,), jnp.int32),
            pltpu.VMEM((_LANES * J,), jnp.int32),
            pltpu.VMEM((2, _JG * _LANES, D), jnp.float32),
            pltpu.VMEM((2, TD, _SUB, _LANES + 1), jnp.float32),
        ] + [pltpu.SemaphoreType.DMA] * 4,
    )


_CHUNK = 800
_NBUF = 2


@functools.cache
def _build_gather_flat(B, D):
    assert B % (_NW * _CHUNK * _NBUF) == 0
    b_per_w = B // _NW
    n_chunks = b_per_w // _CHUNK
    mesh = plsc.VectorSubcoreMesh(core_axis_name="c", subcore_axis_name="s")

    def body(table_hbm, idx_hbm, out_hbm, idx_v, rows_v, *sems):
        wid = lax.axis_index("s") * _NUM_CORES + lax.axis_index("c")
        base = wid * b_per_w

        def issue(g, b):
            off = base + g * _CHUNK
            pltpu.sync_copy(idx_hbm.at[pl.ds(off, _CHUNK)], idx_v.at[b])
            pltpu.async_copy(table_hbm.at[idx_v.at[b]], rows_v.at[b], sems[b])

        def drain_store(g, b):
            pltpu.make_async_copy(
                table_hbm.at[idx_v.at[b]], rows_v.at[b], sems[b]).wait()
            off = base + g * _CHUNK
            pltpu.sync_copy(rows_v.at[b], out_hbm.at[pl.ds(off, _CHUNK)])

        for b in range(_NBUF):
            issue(b, b)

        @pl.loop(0, n_chunks - _NBUF, step=_NBUF)
        def _main(go):
            for b in range(_NBUF):
                drain_store(go + b, b)
                issue(go + b + _NBUF, b)

        for b in range(_NBUF):
            drain_store(n_chunks - _NBUF + b, b)

    return pl.kernel(
        body,
        out_type=jax.ShapeDtypeStruct((B, D), jnp.float32),
        mesh=mesh,
        compiler_params=pltpu.CompilerParams(
            use_tc_tiling_on_sc=False, needs_layout_passes=False),
        scratch_types=[
            pltpu.VMEM((_NBUF, _CHUNK), jnp.int32),
            pltpu.VMEM((_NBUF, _CHUNK, D), jnp.float32),
        ] + [pltpu.SemaphoreType.DMA] * _NBUF,
    )


def kernel(token_ids, weight):
    lead = token_ids.shape
    d = weight.shape[1]
    b = 1
    for s_ in lead:
        b *= s_
    idx = token_ids.reshape((b,)).astype(jnp.int32)
    if (len(lead) == 2 and lead[0] % (_NW * _LANES) == 0 and d % _SUB == 0
            and lead[1] % (2 * _JG) == 0):
        s, j = lead
        p = _build_gather_t(s, j, d)(weight, idx)
        return p.transpose(2, 4, 0, 1, 3).reshape(s, j, d)
    out = _build_gather_flat(b, d)(weight, idx)
    return out.reshape(lead + (d,))


# speedup vs baseline: 1.8302x; 1.0188x over previous
import functools

import jax
import jax.numpy as jnp
from jax import lax
from jax.experimental import pallas as pl
from jax.experimental.pallas import tpu as pltpu
from jax.experimental.pallas import tpu_sc as plsc

_NUM_CORES = 2
_NUM_SUBCORES = 16
_NW = _NUM_CORES * _NUM_SUBCORES
_L = 16
_LANES = 128
_SUB = 8
_JG = 5


@functools.cache
def _build_gather_t(S, J, D):
    assert S % (_NW * _LANES) == 0 and D % _SUB == 0
    assert J % (2 * _JG) == 0
    TD = D // _SUB
    TS = S // _LANES
    nblk = TS // _NW
    ngrp = _LANES // _L
    ngl = J // _JG
    mesh = plsc.VectorSubcoreMesh(core_axis_name="c", subcore_axis_name="s")

    def body(table_hbm, idx_hbm, p_hbm, idx2_v, idxT_v, rows_v, q_v,
             g0, g1, q0, q1):
        gsem = (g0, g1)
        qsem = (q0, q1)
        wid = lax.axis_index("s") * _NUM_CORES + lax.axis_index("c")
        iota = lax.iota(jnp.int32, _L)
        sv = [iota + _L * k for k in range(ngrp)]
        svj = [(iota + _L * k) * J for k in range(ngrp)]

        @pl.loop(0, nblk)
        def _blk(tsb):
            ts = wid * nblk + tsb
            tok0 = ts * _LANES * J

            pltpu.sync_copy(idx_hbm.at[pl.ds(tok0, _LANES * J)], idx2_v)

            @pl.loop(0, J)
            def _tr_idx(j):
                for k in range(ngrp):
                    v = plsc.load_gather(idx2_v, [svj[k] + j])
                    idxT_v[pl.ds(j * _LANES + _L * k, _L)] = v

            def g_issue(grp, b):
                offs = idxT_v.at[pl.ds(grp * _JG * _LANES, _JG * _LANES)]
                pltpu.async_copy(table_hbm.at[offs], rows_v.at[b], gsem[b])

            def g_wait(grp, b):
                offs = idxT_v.at[pl.ds(grp * _JG * _LANES, _JG * _LANES)]
                pltpu.make_async_copy(
                    table_hbm.at[offs], rows_v.at[b], gsem[b]).wait()

            def q_store(j, qb):
                pltpu.async_copy(
                    q_v.at[qb, :, :, pl.ds(0, _LANES)],
                    p_hbm.at[j, :, ts], qsem[qb])

            def q_drain(j, qb):
                pltpu.make_async_copy(
                    q_v.at[qb, :, :, pl.ds(0, _LANES)],
                    p_hbm.at[j, :, ts], qsem[qb]).wait()

            ndg = D // _L
            tdv = [(jnp.full((_L,), dg * _L, jnp.int32) + iota) // _SUB
                   for dg in range(ndg)]
            div = [(jnp.full((_L,), dg * _L, jnp.int32) + iota) % _SUB
                   for dg in range(ndg)]

            def transpose(b, jj, qb):
                @pl.loop(0, _LANES, step=2)
                def _t(s_in):
                    for u in range(2):
                        s = s_in + u
                        row = jj * _LANES + s
                        lane = jnp.full((_L,), s, jnp.int32)
                        for dg in range(ndg):
                            v = rows_v[b, row, pl.ds(dg * _L, _L)]
                            plsc.store_scatter(
                                q_v.at[qb], [tdv[dg], div[dg], lane], v)

            def do_group(grp, b, first):
                g_wait(grp, b)
                for jj in range(_JG):
                    j = grp * _JG + jj
                    qb = jj % 2
                    if not (first and jj < 2):
                        q_drain(j, qb)
                    transpose(b, jj, qb)
                    q_store(j, qb)

            g_issue(0, 0)
            g_issue(1, 1)
            do_group(0, 0, True)
            g_issue(2, 0)
            do_group(1, 1, False)
            g_issue(3, 1)

            @pl.loop(2, ngl - 2, step=2)
            def _main(gv):
                for u in range(2):
                    do_group(gv + u, u, False)
                    g_issue(gv + u + 2, u)

            do_group(ngl - 2, 0, False)
            do_group(ngl - 1, 1, False)
            q_drain(0, 0)
            q_drain(1, 1)

    return pl.kernel(
        body,
        out_type=jax.ShapeDtypeStruct((J, TD, TS, _SUB, _LANES), jnp.float32),
        mesh=mesh,
        compiler_params=pltpu.CompilerParams(
            use_tc_tiling_on_sc=False, needs_layout_passes=False),
        scratch_types=[
            pltpu.VMEM((_LANES * J,), jnp.int32),
            pltpu.VMEM((_LANES * J,), jnp.int32),
            pltpu.VMEM((2, _JG * _LANES, D), jnp.float32),
            pltpu.VMEM((2, TD, _SUB, _LANES + 1), jnp.float32),
        ] + [pltpu.SemaphoreType.DMA] * 4,
    )


_CHUNK = 800
_NBUF = 2


@functools.cache
def _build_gather_flat(B, D):
    assert B % (_NW * _CHUNK * _NBUF) == 0
    b_per_w = B // _NW
    n_chunks = b_per_w // _CHUNK
    mesh = plsc.VectorSubcoreMesh(core_axis_name="c", subcore_axis_name="s")

    def body(table_hbm, idx_hbm, out_hbm, idx_v, rows_v, *sems):
        wid = lax.axis_index("s") * _NUM_CORES + lax.axis_index("c")
        base = wid * b_per_w

        def issue(g, b):
            off = base + g * _CHUNK
            pltpu.sync_copy(idx_hbm.at[pl.ds(off, _CHUNK)], idx_v.at[b])
            pltpu.async_copy(table_hbm.at[idx_v.at[b]], rows_v.at[b], sems[b])

        def drain_store(g, b):
            pltpu.make_async_copy(
                table_hbm.at[idx_v.at[b]], rows_v.at[b], sems[b]).wait()
            off = base + g * _CHUNK
            pltpu.sync_copy(rows_v.at[b], out_hbm.at[pl.ds(off, _CHUNK)])

        for b in range(_NBUF):
            issue(b, b)

        @pl.loop(0, n_chunks - _NBUF, step=_NBUF)
        def _main(go):
            for b in range(_NBUF):
                drain_store(go + b, b)
                issue(go + b + _NBUF, b)

        for b in range(_NBUF):
            drain_store(n_chunks - _NBUF + b, b)

    return pl.kernel(
        body,
        out_type=jax.ShapeDtypeStruct((B, D), jnp.float32),
        mesh=mesh,
        compiler_params=pltpu.CompilerParams(
            use_tc_tiling_on_sc=False, needs_layout_passes=False),
        scratch_types=[
            pltpu.VMEM((_NBUF, _CHUNK), jnp.int32),
            pltpu.VMEM((_NBUF, _CHUNK, D), jnp.float32),
        ] + [pltpu.SemaphoreType.DMA] * _NBUF,
    )


def kernel(token_ids, weight):
    lead = token_ids.shape
    d = weight.shape[1]
    b = 1
    for s_ in lead:
        b *= s_
    idx = token_ids.reshape((b,)).astype(jnp.int32)
    if (len(lead) == 2 and lead[0] % (_NW * _LANES) == 0 and d % _SUB == 0
            and lead[1] % (2 * _JG) == 0):
        s, j = lead
        p = _build_gather_t(s, j, d)(weight, idx)
        return p.transpose(2, 4, 0, 1, 3).reshape(s, j, d)
    out = _build_gather_flat(b, d)(weight, idx)
    return out.reshape(lead + (d,))
